# Initial kernel scaffold; baseline (speedup 1.0000x reference)
import jax
import jax.numpy as jnp
from jax.experimental import pallas as pl


def kernel(grad, sharpness, recurrent_state, inducing_points, input_proj_W, input_proj_b, induce_q_W, induce_k_W, induce_v_W, read_q_W, W_h, W_x_W, W_x_b, peer_query_W, product_keys_A, product_keys_B, expert_W1, expert_b1, expert_W2, expert_b2):
    raise NotImplementedError("write your pallas kernel here")


# speedup vs baseline: 4.7170x; 4.7170x over previous
import functools
import math

import jax
import jax.numpy as jnp
from jax import lax
from jax.experimental import pallas as pl
from jax.experimental.pallas import tpu as pltpu
from jax.experimental.pallas import tpu_sc as plsc

_N = 524288
_D = 64
_M = 32
_PK = 128
_EH = 16
_RD = 32
_RESCALE = 0.1
_SCALE = 1.0 / math.sqrt(_D)

_LANES = 128
_ROWS = _N // _LANES
_R1 = 512
_BT = 2048

_NC = 2
_NS = 16
_NW = _NC * _NS
_TOKW = _N // _NW
_C = 1024
_GRP = _C // _LANES
_CI = _TOKW // _C


def _p1_max_body(g_ref, s_ref, a_ref, b_ref, c_ref, m_ref):
    @pl.when(pl.program_id(0) == 0)
    def _init():
        m_ref[...] = jnp.full((_M, _LANES), -jnp.inf, jnp.float32)

    a = a_ref[...]
    b = b_ref[...]
    c = c_ref[...]

    def row(r, m):
        g = g_ref[pl.ds(r, 1), :]
        s = s_ref[pl.ds(r, 1), :]
        return jnp.maximum(m, a * g + b * s + c)

    m_ref[...] = lax.fori_loop(0, _R1, row, m_ref[...])


def _p1_sum_body(g_ref, s_ref, a_ref, b_ref, c_ref, mx_ref,
                 s_out, eg_out, es_out):
    @pl.when(pl.program_id(0) == 0)
    def _init():
        z = jnp.zeros((_M, _LANES), jnp.float32)
        s_out[...] = z
        eg_out[...] = z
        es_out[...] = z

    a = a_ref[...]
    b = b_ref[...]
    c = c_ref[...] - mx_ref[...]

    def row(r, carry):
        acc_s, acc_g, acc_sh = carry
        g = g_ref[pl.ds(r, 1), :]
        s = s_ref[pl.ds(r, 1), :]
        e = jnp.exp(a * g + b * s + c)
        return (acc_s + e, acc_g + e * g, acc_sh + e * s)

    acc_s, acc_g, acc_sh = lax.fori_loop(
        0, _R1, row, (s_out[...], eg_out[...], es_out[...]))
    s_out[...] = acc_s
    eg_out[...] = acc_g
    es_out[...] = acc_sh


def _p2_body(g_ref, s_ref, h_ref, wht_ref, c32_ref, wqh_ref, pm_ref,
             c64_ref, pka_ref, pkb_ref, hn_ref, idx_ref):
    g = g_ref[...]
    s = s_ref[...]
    c32 = c32_ref[...]
    hn = jnp.tanh(
        jnp.dot(h_ref[...], wht_ref[...], preferred_element_type=jnp.float32)
        + g * c32[0:1] + s * c32[1:2] + c32[2:3])
    l2 = g * c32[3:4] + s * c32[4:5] + c32[5:6]
    l2 = l2 - jnp.max(l2, axis=1, keepdims=True)
    e2 = jnp.exp(l2)
    ra = e2 / jnp.sum(e2, axis=1, keepdims=True)
    c64 = c64_ref[...]
    q = (jnp.dot(hn, wqh_ref[...], preferred_element_type=jnp.float32)
         + jnp.dot(ra, pm_ref[...], preferred_element_type=jnp.float32)
         + g * c64[0:1] + s * c64[1:2])
    sa = jnp.dot(q[:, :_D // 2], pka_ref[...],
                 preferred_element_type=jnp.float32)
    sb = jnp.dot(q[:, _D // 2:], pkb_ref[...],
                 preferred_element_type=jnp.float32)
    iota = lax.broadcasted_iota(jnp.int32, (_BT, _PK), 1)
    big = jnp.int32(2 ** 30)
    ia = jnp.min(jnp.where(sa >= jnp.max(sa, axis=1, keepdims=True),
                           iota, big), axis=1, keepdims=True)
    ib = jnp.min(jnp.where(sb >= jnp.max(sb, axis=1, keepdims=True),
                           iota, big), axis=1, keepdims=True)
    ei = jnp.clip(ia * _PK + ib, 0, _PK * _PK - 1)
    hn_ref[...] = hn
    idx_ref[...] = ei.astype(jnp.int32)


def _col16(k):
    return jnp.full((16,), k, jnp.int32)


def _p3_body(g_hbm, idx_hbm, tab_hbm, out_hbm, idx_v, rows_v, g_v, out_v,
             sem):
    wid = lax.axis_index("s") * _NC + lax.axis_index("c")
    base0 = wid * _TOKW

    def chunk(ci, carry):
        base = pl.multiple_of(base0 + ci * _C, _C)
        row0 = pl.multiple_of(base // _LANES, _GRP)
        pltpu.sync_copy(idx_hbm.at[pl.ds(row0, _GRP)], idx_v)
        pltpu.sync_copy(g_hbm.at[pl.ds(base, _C)], g_v)
        dmas = []
        for j in range(_GRP):
            dmas.append(pltpu.async_copy(
                tab_hbm.at[idx_v.at[j]],
                rows_v.at[pl.ds(j * _LANES, _LANES)], sem))
        for d in dmas:
            d.wait()

        def tok16(i, c2):
            b16 = i * 16
            gv = g_v[pl.ds(b16, 16)]
            ridx = b16 + lax.iota(jnp.int32, 16)
            acc = plsc.load_gather(rows_v, [ridx, _col16(48)])
            for k in range(_EH):
                w1 = plsc.load_gather(rows_v, [ridx, _col16(k)])
                b1 = plsc.load_gather(rows_v, [ridx, _col16(_EH + k)])
                w2 = plsc.load_gather(rows_v, [ridx, _col16(2 * _EH + k)])
                z = jnp.maximum(w1 * gv + b1, jnp.float32(0.0))
                acc = acc + w2 * z
            out_v[pl.ds(b16, 16)] = gv + jnp.float32(_RESCALE) * acc
            return c2

        lax.fori_loop(0, _C // 16, tok16, 0)
        pltpu.sync_copy(out_v, out_hbm.at[pl.ds(base, _C)])
        return carry

    lax.fori_loop(0, _CI, chunk, 0)


@functools.lru_cache(maxsize=1)
def _p3_call():
    mesh = plsc.VectorSubcoreMesh(
        core_axis_name="c", subcore_axis_name="s",
        num_cores=_NC, num_subcores=_NS)
    return pl.kernel(
        _p3_body,
        out_type=jax.ShapeDtypeStruct((_N,), jnp.float32),
        mesh=mesh,
        scratch_types=[
            pltpu.VMEM((_GRP, _LANES), jnp.int32),
            pltpu.VMEM((_C, _D), jnp.float32),
            pltpu.VMEM((_C,), jnp.float32),
            pltpu.VMEM((_C,), jnp.float32),
            pltpu.SemaphoreType.DMA,
        ],
        compiler_params=pltpu.CompilerParams(
            needs_layout_passes=False, use_tc_tiling_on_sc=False),
    )


def kernel(grad, sharpness, recurrent_state, inducing_points, input_proj_W,
           input_proj_b, induce_q_W, induce_k_W, induce_v_W, read_q_W, W_h,
           W_x_W, W_x_b, peer_query_W, product_keys_A, product_keys_B,
           expert_W1, expert_b1, expert_W2, expert_b2):
    f32 = jnp.float32
    g = grad.astype(f32)
    s = sharpness.astype(f32)
    h = recurrent_state.astype(f32)

    wg = input_proj_W[:, 0]
    ws = input_proj_W[:, 1]
    bx = input_proj_b
    iq = inducing_points @ induce_q_W.T
    kg = induce_k_W @ wg
    ks = induce_k_W @ ws
    kb = induce_k_W @ bx
    a1 = (_SCALE * (iq @ kg)).reshape(_M, 1).astype(f32)
    b1 = (_SCALE * (iq @ ks)).reshape(_M, 1).astype(f32)
    c1 = (_SCALE * (iq @ kb)).reshape(_M, 1).astype(f32)

    g2d = g.reshape(_ROWS, _LANES)
    s2d = s.reshape(_ROWS, _LANES)
    nsteps = _ROWS // _R1
    csts = pl.BlockSpec((_M, 1), lambda i: (0, 0))
    acc_spec = pl.BlockSpec((_M, _LANES), lambda i: (0, 0))
    gs_spec = pl.BlockSpec((_R1, _LANES), lambda i: (i, 0))
    arb = pltpu.CompilerParams(dimension_semantics=("arbitrary",))

    m_part = pl.pallas_call(
        _p1_max_body,
        grid=(nsteps,),
        in_specs=[gs_spec, gs_spec, csts, csts, csts],
        out_specs=acc_spec,
        out_shape=jax.ShapeDtypeStruct((_M, _LANES), f32),
        compiler_params=arb,
    )(g2d, s2d, a1, b1, c1)
    mcol = jnp.max(m_part, axis=1, keepdims=True)

    s_p, eg_p, es_p = pl.pallas_call(
        _p1_sum_body,
        grid=(nsteps,),
        in_specs=[gs_spec, gs_spec, csts, csts, csts, csts],
        out_specs=[acc_spec, acc_spec, acc_spec],
        out_shape=[jax.ShapeDtypeStruct((_M, _LANES), f32)] * 3,
        compiler_params=arb,
    )(g2d, s2d, a1, b1, c1, mcol)

    ssum = jnp.sum(s_p, axis=1)
    gh = jnp.sum(eg_p, axis=1) / ssum
    sh = jnp.sum(es_p, axis=1) / ssum
    vg = induce_v_W @ wg
    vs = induce_v_W @ ws
    vb = induce_v_W @ bx
    i_up = gh[:, None] * vg[None, :] + sh[:, None] * vs[None, :] + vb[None, :]

    rg = read_q_W @ wg
    rs = read_q_W @ ws
    rb = read_q_W @ bx
    a2 = _SCALE * (i_up @ rg)
    b2 = _SCALE * (i_up @ rs)
    c2 = _SCALE * (i_up @ rb)
    z32 = jnp.zeros((_RD,), f32)
    c32 = jnp.stack([W_x_W[:, 0], W_x_W[:, 1], W_x_b, a2, b2, c2, z32, z32])
    z64 = jnp.zeros((_D,), f32)
    c64 = jnp.stack([peer_query_W[:, _RD + _D], peer_query_W[:, _RD + _D + 1],
                     z64, z64, z64, z64, z64, z64])
    wht = W_h.T
    wqh = peer_query_W[:, :_RD].T
    pm = i_up @ peer_query_W[:, _RD:_RD + _D].T
    pka = product_keys_A.T
    pkb = product_keys_B.T

    gcol = g.reshape(_N, 1)
    scol = s.reshape(_N, 1)
    cst = lambda shape: pl.BlockSpec(shape, lambda i: (0, 0))
    hn, eidx = pl.pallas_call(
        _p2_body,
        grid=(_N // _BT,),
        in_specs=[
            pl.BlockSpec((_BT, 1), lambda i: (i, 0)),
            pl.BlockSpec((_BT, 1), lambda i: (i, 0)),
            pl.BlockSpec((_BT, _RD), lambda i: (i, 0)),
            cst((_RD, _RD)),
            cst((8, _RD)),
            cst((_RD, _D)),
            cst((_M, _D)),
            cst((8, _D)),
            cst((_D // 2, _PK)),
            cst((_D // 2, _PK)),
        ],
        out_specs=[
            pl.BlockSpec((_BT, _RD), lambda i: (i, 0)),
            pl.BlockSpec((_BT, 1), lambda i: (i, 0)),
        ],
        out_shape=[
            jax.ShapeDtypeStruct((_N, _RD), f32),
            jax.ShapeDtypeStruct((_N, 1), jnp.int32),
        ],
    )(gcol, scol, h, wht, c32, wqh, pm, c64, pka, pkb)

    tab = jnp.concatenate(
        [expert_W1[:, :, 0], expert_b1, expert_W2[:, 0, :], expert_b2,
         jnp.zeros((_PK * _PK, 15), f32)], axis=1)
    eidx2d = eidx.reshape(_N // _LANES, _LANES)
    smart = _p3_call()(g, eidx2d, tab)
    return smart.reshape(grad.shape), hn


# speedup vs baseline: 14.8250x; 3.1429x over previous
import functools
import math

import jax
import jax.numpy as jnp
from jax import lax
from jax.experimental import pallas as pl
from jax.experimental.pallas import tpu as pltpu
from jax.experimental.pallas import tpu_sc as plsc

_N = 524288
_D = 64
_M = 32
_PK = 128
_EH = 16
_RD = 32
_RESCALE = 0.1
_SCALE = 1.0 / math.sqrt(_D)

_LANES = 128
_ROWS = _N // _LANES
_R1A = 512
_R1B = 128
_BT = 2048

_NC = 2
_NS = 16
_NW = _NC * _NS
_TOKW = _N // _NW
_C = 512
_GRP = _C // _LANES
_CI = _TOKW // _C


def _p1_max_body(g_ref, s_ref, a_ref, b_ref, m_ref):
    @pl.when(pl.program_id(0) == 0)
    def _init():
        m_ref[...] = jnp.full((_M, _LANES), -jnp.inf, jnp.float32)

    gb = g_ref[...]
    sb = s_ref[...]
    for q in range(_M):
        t = a_ref[q:q + 1, 0:1] * gb + b_ref[q:q + 1, 0:1] * sb
        mq = jnp.max(t, axis=0, keepdims=True)
        m_ref[q:q + 1, :] = jnp.maximum(m_ref[q:q + 1, :], mq)


def _p1_sum_body(g_ref, s_ref, a_ref, b_ref, d_ref, s_out, eg_out, es_out):
    @pl.when(pl.program_id(0) == 0)
    def _init():
        z = jnp.zeros((_M, _LANES), jnp.float32)
        s_out[...] = z
        eg_out[...] = z
        es_out[...] = z

    gb = g_ref[...]
    sb = s_ref[...]
    for q in range(_M):
        e = jnp.exp(a_ref[q:q + 1, 0:1] * gb + b_ref[q:q + 1, 0:1] * sb
                    - d_ref[q:q + 1, 0:1])
        s_out[q:q + 1, :] += jnp.sum(e, axis=0, keepdims=True)
        eg_out[q:q + 1, :] += jnp.sum(e * gb, axis=0, keepdims=True)
        es_out[q:q + 1, :] += jnp.sum(e * sb, axis=0, keepdims=True)


def _p2_body(g_ref, s_ref, h_ref, wh_ref, cc_ref, wq2_ref, pmt_ref,
             qc_ref, pka_ref, pkb_ref, hn_ref, idx_ref):
    g = g_ref[...]
    s = s_ref[...]
    cc = cc_ref[...]
    hnt = jnp.tanh(
        jnp.dot(wh_ref[...], h_ref[...], preferred_element_type=jnp.float32)
        + cc[:, 0:1] * g + cc[:, 1:2] * s + cc[:, 2:3])
    l2 = cc[:, 3:4] * g + cc[:, 4:5] * s + cc[:, 5:6]
    l2 = l2 - jnp.max(l2, axis=0, keepdims=True)
    e2 = jnp.exp(l2)
    ra = e2 / jnp.sum(e2, axis=0, keepdims=True)
    qc = qc_ref[...]
    qt = (jnp.dot(wq2_ref[...], hnt, preferred_element_type=jnp.float32)
          + jnp.dot(pmt_ref[...], ra, preferred_element_type=jnp.float32)
          + qc[:, 0:1] * g + qc[:, 1:2] * s)
    sa = jnp.dot(pka_ref[...], qt[:_D // 2, :],
                 preferred_element_type=jnp.float32)
    sb = jnp.dot(pkb_ref[...], qt[_D // 2:, :],
                 preferred_element_type=jnp.float32)
    iota = lax.broadcasted_iota(jnp.int32, (_PK, _BT), 0).astype(jnp.float32)
    big = jnp.float32(65536.0)
    ia = jnp.min(jnp.where(sa >= jnp.max(sa, axis=0, keepdims=True),
                           iota, big), axis=0, keepdims=True)
    ib = jnp.min(jnp.where(sb >= jnp.max(sb, axis=0, keepdims=True),
                           iota, big), axis=0, keepdims=True)
    ei = jnp.clip(ia * jnp.float32(_PK) + ib, 0.0, float(_PK * _PK - 1))
    hn_ref[...] = hnt.T
    idx_ref[...] = ei.astype(jnp.int32)


def _col16(k):
    return jnp.full((16,), k, jnp.int32)


def _p3_body(g_hbm, idx_hbm, tab_hbm, out_hbm, idx_v, rows_v, g_v, out_v,
             sem0, sem1):
    wid = lax.axis_index("s") * _NC + lax.axis_index("c")
    base0 = wid * _TOKW
    sems = (sem0, sem1)

    def fire(ch, b):
        base = pl.multiple_of(base0 + ch * _C, _C)
        row0 = pl.multiple_of(base // _LANES, _GRP)
        pltpu.sync_copy(idx_hbm.at[pl.ds(row0, _GRP)], idx_v.at[b])
        pltpu.sync_copy(g_hbm.at[pl.ds(base, _C)], g_v.at[b])
        for j in range(_GRP):
            pltpu.async_copy(
                tab_hbm.at[idx_v.at[b].at[j]],
                rows_v.at[b].at[pl.ds(j * _LANES, _LANES)], sems[b])

    def drain(b):
        pltpu.make_async_copy(
            tab_hbm.at[pl.ds(0, _C)], rows_v.at[b], sems[b]).wait()

    def compute(ch, b):
        rows = rows_v.at[b]
        gv_ref = g_v.at[b]
        ov = out_v.at[b]

        def tok16(i, c2):
            b16 = i * 16
            gv = gv_ref[pl.ds(b16, 16)]
            ridx = b16 + lax.iota(jnp.int32, 16)
            acc = plsc.load_gather(rows, [ridx, _col16(48)])
            for k in range(_EH):
                w1 = plsc.load_gather(rows, [ridx, _col16(k)])
                b1 = plsc.load_gather(rows, [ridx, _col16(_EH + k)])
                w2 = plsc.load_gather(rows, [ridx, _col16(2 * _EH + k)])
                z = jnp.maximum(w1 * gv + b1, jnp.float32(0.0))
                acc = acc + w2 * z
            ov[pl.ds(b16, 16)] = gv + jnp.float32(_RESCALE) * acc
            return c2

        lax.fori_loop(0, _C // 16, tok16, 0)
        base = pl.multiple_of(base0 + ch * _C, _C)
        pltpu.sync_copy(ov, out_hbm.at[pl.ds(base, _C)])

    fire(0, 0)
    fire(1, 1)

    def step(i2, carry):
        for b in range(2):
            ch = i2 * 2 + b
            drain(b)
            compute(ch, b)

            @pl.when(ch + 2 < _CI)
            def _():
                fire(ch + 2, b)
        return carry

    lax.fori_loop(0, _CI // 2, step, 0)


@functools.lru_cache(maxsize=1)
def _p3_call():
    mesh = plsc.VectorSubcoreMesh(
        core_axis_name="c", subcore_axis_name="s",
        num_cores=_NC, num_subcores=_NS)
    return pl.kernel(
        _p3_body,
        out_type=jax.ShapeDtypeStruct((_N,), jnp.float32),
        mesh=mesh,
        scratch_types=[
            pltpu.VMEM((2, _GRP, _LANES), jnp.int32),
            pltpu.VMEM((2, _C, _D), jnp.float32),
            pltpu.VMEM((2, _C), jnp.float32),
            pltpu.VMEM((2, _C), jnp.float32),
            pltpu.SemaphoreType.DMA,
            pltpu.SemaphoreType.DMA,
        ],
        compiler_params=pltpu.CompilerParams(
            needs_layout_passes=False, use_tc_tiling_on_sc=False),
    )


def kernel(grad, sharpness, recurrent_state, inducing_points, input_proj_W,
           input_proj_b, induce_q_W, induce_k_W, induce_v_W, read_q_W, W_h,
           W_x_W, W_x_b, peer_query_W, product_keys_A, product_keys_B,
           expert_W1, expert_b1, expert_W2, expert_b2):
    f32 = jnp.float32
    g = grad.astype(f32)
    s = sharpness.astype(f32)
    h = recurrent_state.astype(f32)

    wg = input_proj_W[:, 0]
    ws = input_proj_W[:, 1]
    bx = input_proj_b
    iq = inducing_points @ induce_q_W.T
    kg = induce_k_W @ wg
    ks = induce_k_W @ ws
    kb = induce_k_W @ bx
    a1 = (_SCALE * (iq @ kg)).reshape(_M, 1).astype(f32)
    b1 = (_SCALE * (iq @ ks)).reshape(_M, 1).astype(f32)

    g2d = g.reshape(_ROWS, _LANES)
    s2d = s.reshape(_ROWS, _LANES)
    csts = pl.BlockSpec((_M, 1), lambda i: (0, 0))
    acc_spec = pl.BlockSpec((_M, _LANES), lambda i: (0, 0))
    arb = pltpu.CompilerParams(dimension_semantics=("arbitrary",))

    gs_a = pl.BlockSpec((_R1A, _LANES), lambda i: (i, 0))
    m_part = pl.pallas_call(
        _p1_max_body,
        grid=(_ROWS // _R1A,),
        in_specs=[gs_a, gs_a, csts, csts],
        out_specs=acc_spec,
        out_shape=jax.ShapeDtypeStruct((_M, _LANES), f32),
        compiler_params=arb,
    )(g2d, s2d, a1, b1)
    mcol = jnp.max(m_part, axis=1, keepdims=True)

    gs_b = pl.BlockSpec((_R1B, _LANES), lambda i: (i, 0))
    s_p, eg_p, es_p = pl.pallas_call(
        _p1_sum_body,
        grid=(_ROWS // _R1B,),
        in_specs=[gs_b, gs_b, csts, csts, csts],
        out_specs=[acc_spec, acc_spec, acc_spec],
        out_shape=[jax.ShapeDtypeStruct((_M, _LANES), f32)] * 3,
        compiler_params=arb,
    )(g2d, s2d, a1, b1, mcol)

    ssum = jnp.sum(s_p, axis=1)
    gh = jnp.sum(eg_p, axis=1) / ssum
    sh = jnp.sum(es_p, axis=1) / ssum
    vg = induce_v_W @ wg
    vs = induce_v_W @ ws
    vb = induce_v_W @ bx
    i_up = gh[:, None] * vg[None, :] + sh[:, None] * vs[None, :] + vb[None, :]

    rg = read_q_W @ wg
    rs = read_q_W @ ws
    rb = read_q_W @ bx
    a2 = _SCALE * (i_up @ rg)
    b2 = _SCALE * (i_up @ rs)
    c2 = _SCALE * (i_up @ rb)
    cc = jnp.stack([W_x_W[:, 0], W_x_W[:, 1], W_x_b, a2, b2, c2,
                    jnp.zeros((_RD,), f32), jnp.zeros((_RD,), f32)], axis=1)
    qc = jnp.stack([peer_query_W[:, _RD + _D], peer_query_W[:, _RD + _D + 1],
                    jnp.zeros((_D,), f32), jnp.zeros((_D,), f32)], axis=1)
    wq2 = peer_query_W[:, :_RD]
    pmt = peer_query_W[:, _RD:_RD + _D] @ i_up.T

    grow = g.reshape(1, _N)
    srow = s.reshape(1, _N)
    ht = h.T
    cst = lambda shape: pl.BlockSpec(shape, lambda i: (0, 0))
    hn, eidx = pl.pallas_call(
        _p2_body,
        grid=(_N // _BT,),
        in_specs=[
            pl.BlockSpec((1, _BT), lambda i: (0, i)),
            pl.BlockSpec((1, _BT), lambda i: (0, i)),
            pl.BlockSpec((_RD, _BT), lambda i: (0, i)),
            cst((_RD, _RD)),
            cst((_RD, 8)),
            cst((_D, _RD)),
            cst((_D, _M)),
            cst((_D, 4)),
            cst((_PK, _D // 2)),
            cst((_PK, _D // 2)),
        ],
        out_specs=[
            pl.BlockSpec((_BT, _RD), lambda i: (i, 0)),
            pl.BlockSpec((1, _BT), lambda i: (0, i)),
        ],
        out_shape=[
            jax.ShapeDtypeStruct((_N, _RD), f32),
            jax.ShapeDtypeStruct((1, _N), jnp.int32),
        ],
    )(grow, srow, ht, W_h, cc, wq2, pmt, qc, product_keys_A, product_keys_B)

    tab = jnp.concatenate(
        [expert_W1[:, :, 0], expert_b1, expert_W2[:, 0, :], expert_b2,
         jnp.zeros((_PK * _PK, 15), f32)], axis=1)
    eidx2d = eidx.reshape(_N // _LANES, _LANES)
    smart = _p3_call()(g, eidx2d, tab)
    return smart.reshape(grad.shape), hn
